# Initial kernel scaffold; baseline (speedup 1.0000x reference)
import jax
import jax.numpy as jnp
from jax.experimental import pallas as pl


def kernel(x, table):
    raise NotImplementedError("write your pallas kernel here")


# speedup vs baseline: 9.5498x; 9.5498x over previous
import functools

import jax
import jax.numpy as jnp
from jax import lax
from jax.experimental import pallas as pl
from jax.experimental.pallas import tpu as pltpu
from jax.experimental.pallas import tpu_sc as plsc

_CHUNK = 1024


@functools.lru_cache(maxsize=None)
def _make_gather(n_idx: int, d: int):
    info = plsc.get_sparse_core_info()
    nc, ns = info.num_cores, info.num_subcores
    nw = nc * ns
    assert n_idx % nw == 0
    per_w = n_idx // nw
    assert per_w % _CHUNK == 0
    n_chunks = per_w // _CHUNK

    mesh = plsc.VectorSubcoreMesh(core_axis_name="c", subcore_axis_name="s")

    @functools.partial(
        pl.kernel,
        mesh=mesh,
        out_type=jax.ShapeDtypeStruct((n_idx, d), jnp.float32),
        scratch_types=[
            pltpu.VMEM((_CHUNK,), jnp.int32),
            pltpu.VMEM((_CHUNK, d), jnp.float32),
            pltpu.SemaphoreType.DMA,
        ],
        compiler_params=pltpu.CompilerParams(use_tc_tiling_on_sc=False),
    )
    def gather_kernel(idx_hbm, table_hbm, out_hbm, idx_v, rows_v, sem):
        wid = lax.axis_index("s") * nc + lax.axis_index("c")
        base = wid * per_w

        def body(i, carry):
            off = base + i * _CHUNK
            pltpu.sync_copy(idx_hbm.at[pl.ds(off, _CHUNK)], idx_v)
            pltpu.async_copy(table_hbm.at[idx_v], rows_v, sem).wait()
            pltpu.sync_copy(rows_v, out_hbm.at[pl.ds(off, _CHUNK)])
            return carry

        lax.fori_loop(0, n_chunks, body, 0)

    return gather_kernel


def kernel(x, table):
    b, l = x.shape
    d = table.shape[1]
    idx = x.reshape(-1).astype(jnp.int32)
    out = _make_gather(idx.shape[0], d)(idx, table)
    return out.reshape(b, l * d)


# speedup vs baseline: 10.5166x; 1.1012x over previous
import functools

import jax
import jax.numpy as jnp
from jax import lax
from jax.experimental import pallas as pl
from jax.experimental.pallas import tpu as pltpu
from jax.experimental.pallas import tpu_sc as plsc

_CHUNK = 1024


@functools.lru_cache(maxsize=None)
def _make_gather(n_idx: int, d: int):
    info = plsc.get_sparse_core_info()
    nc, ns = info.num_cores, info.num_subcores
    nw = nc * ns
    assert n_idx % nw == 0
    per_w = n_idx // nw
    assert per_w % (2 * _CHUNK) == 0
    n_chunks = per_w // _CHUNK
    half = n_chunks // 2

    mesh = plsc.VectorSubcoreMesh(core_axis_name="c", subcore_axis_name="s")

    @functools.partial(
        pl.kernel,
        mesh=mesh,
        out_type=jax.ShapeDtypeStruct((n_idx, d), jnp.float32),
        scratch_types=[
            pltpu.VMEM((_CHUNK,), jnp.int32),
            pltpu.VMEM((_CHUNK,), jnp.int32),
            pltpu.VMEM((_CHUNK, d), jnp.float32),
            pltpu.VMEM((_CHUNK, d), jnp.float32),
            pltpu.SemaphoreType.DMA,
            pltpu.SemaphoreType.DMA,
            pltpu.SemaphoreType.DMA,
            pltpu.SemaphoreType.DMA,
            pltpu.SemaphoreType.DMA,
            pltpu.SemaphoreType.DMA,
        ],
        compiler_params=pltpu.CompilerParams(use_tc_tiling_on_sc=False),
    )
    def gather_kernel(idx_hbm, table_hbm, out_hbm, idx0, idx1, rows0, rows1,
                      si0, si1, sg0, sg1, ss0, ss1):
        wid = lax.axis_index("s") * nc + lax.axis_index("c")
        base = wid * per_w

        def idx_load(g, buf, sem):
            pltpu.async_copy(idx_hbm.at[pl.ds(base + g * _CHUNK, _CHUNK)],
                             buf, sem)

        def store(g, buf, sem):
            pltpu.async_copy(buf, out_hbm.at[pl.ds(base + g * _CHUNK, _CHUNK)],
                             sem)

        idx_load(0, idx0, si0)
        idx_load(1, idx1, si1)
        pltpu.make_async_copy(idx_hbm.at[pl.ds(base, _CHUNK)], idx0, si0).wait()
        pltpu.async_copy(table_hbm.at[idx0], rows0, sg0)

        def body(t, carry):
            @pl.when(t >= 1)
            def _():
                pltpu.make_async_copy(
                    rows1, out_hbm.at[pl.ds(base, _CHUNK)], ss1).wait()
            pltpu.make_async_copy(
                idx_hbm.at[pl.ds(base, _CHUNK)], idx1, si1).wait()
            pltpu.async_copy(table_hbm.at[idx1], rows1, sg1)
            pltpu.make_async_copy(table_hbm.at[idx0], rows0, sg0).wait()
            store(2 * t, rows0, ss0)

            @pl.when(t < half - 1)
            def _():
                idx_load(2 * t + 2, idx0, si0)

            @pl.when(t < half - 1)
            def _():
                pltpu.make_async_copy(
                    rows0, out_hbm.at[pl.ds(base, _CHUNK)], ss0).wait()
                pltpu.make_async_copy(
                    idx_hbm.at[pl.ds(base, _CHUNK)], idx0, si0).wait()
                pltpu.async_copy(table_hbm.at[idx0], rows0, sg0)
            pltpu.make_async_copy(table_hbm.at[idx1], rows1, sg1).wait()
            store(2 * t + 1, rows1, ss1)

            @pl.when(t < half - 1)
            def _():
                idx_load(2 * t + 3, idx1, si1)

            return carry

        lax.fori_loop(0, half, body, 0)

        pltpu.make_async_copy(rows0, out_hbm.at[pl.ds(base, _CHUNK)], ss0).wait()
        pltpu.make_async_copy(rows1, out_hbm.at[pl.ds(base, _CHUNK)], ss1).wait()

    return gather_kernel


def kernel(x, table):
    b, l = x.shape
    d = table.shape[1]
    idx = x.reshape(-1).astype(jnp.int32)
    out = _make_gather(idx.shape[0], d)(idx, table)
    return out.reshape(b, l * d)
